# Initial kernel scaffold; baseline (speedup 1.0000x reference)
import jax
import jax.numpy as jnp
from jax.experimental import pallas as pl


def kernel(x, edge_index, Wq1, bq1, Wv1, bv1, Wl, bl, Wl2, bl2):
    raise NotImplementedError("write your pallas kernel here")


# speedup vs baseline: 1.1940x; 1.1940x over previous
import functools

import jax
import jax.numpy as jnp
from jax.experimental import pallas as pl
from jax.experimental.pallas import tpu as pltpu

N = 10000
E = 320000
D = 128


NB = 10
BR = N // NB


HI = jax.lax.Precision.HIGHEST


def _mm(a, b):
    return jnp.dot(a, b, precision=HI)


def _ln(q):
    mu = jnp.mean(q, axis=1, keepdims=True)
    var = jnp.mean((q - mu) ** 2, axis=1, keepdims=True)
    return (q - mu) / jnp.sqrt(var + 1e-5)


def _dense_a_body(x_ref, wq_ref, bq_ref, wv_ref, bv_ref, wl_ref, bl_ref,
                  qn_ref, v_ref, v2_ref, hmax_ref, hmin_ref):
    i = pl.program_id(0)
    x = x_ref[...]
    Q = jnp.maximum(_mm(x, wq_ref[...]) + bq_ref[...], 0.0)
    v_ref[...] = _mm(x, wv_ref[...]) + bv_ref[...]
    Qn = jnp.maximum(_ln(Q), 0.0)
    qn_ref[...] = Qn
    H = jnp.sum(Qn, axis=1, keepdims=True) ** 2
    hmx = jnp.max(H).reshape(1, 1)
    hmn = jnp.min(H).reshape(1, 1)

    @pl.when(i == 0)
    def _init():
        hmax_ref[...] = hmx
        hmin_ref[...] = hmn

    @pl.when(i > 0)
    def _acc():
        hmax_ref[...] = jnp.maximum(hmax_ref[...], hmx)
        hmin_ref[...] = jnp.minimum(hmin_ref[...], hmn)
    v2_ref[...] = _mm(x, wl_ref[...]) + bl_ref[...]


def _dense_b_body(qn_ref, v_ref, hmax_ref, hmin_ref, cos_ref, sin_ref,
                  ctv_ref, stv_ref, csum_ref, ssum_ref):
    i = pl.program_id(0)
    m = (jnp.max(hmax_ref[...]) - jnp.min(hmin_ref[...])) * 2.0
    Qn = qn_ref[...]
    H = jnp.sum(Qn, axis=1, keepdims=True) ** 2
    H2 = (jnp.pi / (2.0 * m)) * H
    cosQ = Qn * jnp.cos(H2)
    sinQ = Qn * jnp.sin(H2)
    cos_ref[...] = cosQ
    sin_ref[...] = sinQ
    V = v_ref[...]
    dimn = (((0,), (0,)), ((), ()))
    ctv = jax.lax.dot_general(cosQ, V, dimn, precision=HI)
    stv = jax.lax.dot_general(sinQ, V, dimn, precision=HI)
    csum = jnp.sum(cosQ, axis=0, keepdims=True)
    ssum = jnp.sum(sinQ, axis=0, keepdims=True)

    @pl.when(i == 0)
    def _init():
        ctv_ref[...] = ctv
        stv_ref[...] = stv
        csum_ref[...] = csum
        ssum_ref[...] = ssum

    @pl.when(i > 0)
    def _acc():
        ctv_ref[...] += ctv
        stv_ref[...] += stv
        csum_ref[...] += csum
        ssum_ref[...] += ssum


def _dense_c_body(cos_ref, sin_ref, ctv_ref, stv_ref, csum_ref, ssum_ref,
                  fo_ref):
    cosQ = cos_ref[...]
    sinQ = sin_ref[...]
    out_f = _mm(cosQ, ctv_ref[...]) + _mm(sinQ, stv_ref[...])
    dimv = (((1,), (1,)), ((), ()))
    norm = (jax.lax.dot_general(cosQ, csum_ref[...], dimv, precision=HI)
            + jax.lax.dot_general(sinQ, ssum_ref[...], dimv, precision=HI) + 1e-10)
    fo_ref[...] = out_f / norm


def _row_spec(w=D):
    return pl.BlockSpec((BR, w), lambda i: (i, 0))


def _full_spec(shape):
    return pl.BlockSpec(shape, lambda i: tuple(0 for _ in shape))


@jax.jit
def _dense(x, Wq1, bq1, Wv1, bv1, Wl, bl, Wl2, bl2):
    wspec = _full_spec((D, D))
    bspec = _full_spec((1, D))
    Qn, V, V2, hmax, hmin = pl.pallas_call(
        _dense_a_body,
        grid=(NB,),
        in_specs=[_row_spec()] + [wspec, bspec] * 3,
        out_specs=(_row_spec(), _row_spec(), _row_spec(),
                   _full_spec((1, 1)), _full_spec((1, 1))),
        out_shape=(
            jax.ShapeDtypeStruct((N, D), jnp.float32),
            jax.ShapeDtypeStruct((N, D), jnp.float32),
            jax.ShapeDtypeStruct((N, D), jnp.float32),
            jax.ShapeDtypeStruct((1, 1), jnp.float32),
            jax.ShapeDtypeStruct((1, 1), jnp.float32),
        ),
    )(x, Wq1, bq1.reshape(1, D), Wv1, bv1.reshape(1, D), Wl, bl.reshape(1, D))

    cosQ, sinQ, ctv, stv, csum, ssum = pl.pallas_call(
        _dense_b_body,
        grid=(NB,),
        in_specs=[_row_spec(), _row_spec(),
                  _full_spec((1, 1)), _full_spec((1, 1))],
        out_specs=(_row_spec(), _row_spec(), _full_spec((D, D)),
                   _full_spec((D, D)), _full_spec((1, D)),
                   _full_spec((1, D))),
        out_shape=(
            jax.ShapeDtypeStruct((N, D), jnp.float32),
            jax.ShapeDtypeStruct((N, D), jnp.float32),
            jax.ShapeDtypeStruct((D, D), jnp.float32),
            jax.ShapeDtypeStruct((D, D), jnp.float32),
            jax.ShapeDtypeStruct((1, D), jnp.float32),
            jax.ShapeDtypeStruct((1, D), jnp.float32),
        ),
    )(Qn, V, hmax, hmin)

    former_out = pl.pallas_call(
        _dense_c_body,
        grid=(NB,),
        in_specs=[_row_spec(), _row_spec(), _full_spec((D, D)),
                  _full_spec((D, D)), _full_spec((1, D)),
                  _full_spec((1, D))],
        out_specs=_row_spec(),
        out_shape=jax.ShapeDtypeStruct((N, D), jnp.float32),
    )(cosQ, sinQ, ctv, stv, csum, ssum)
    return former_out, V2


def kernel(x, edge_index, Wq1, bq1, Wv1, bv1, Wl, bl, Wl2, bl2):
    eps = 1e-10
    former_out, V2 = _dense(x, Wq1, bq1, Wv1, bv1, Wl, bl, Wl2, bl2)
    Q2 = jax.nn.relu(x @ Wl2 + bl2)
    mu2 = jnp.mean(Q2, axis=-1, keepdims=True)
    var2 = jnp.var(Q2, axis=-1, keepdims=True)
    h2n = (Q2 - mu2) / jnp.sqrt(var2 + 1e-5)
    h = jnp.sum(h2n, axis=1, keepdims=True) ** 2
    h = h[:, 0]
    src = edge_index[0].astype(jnp.int32)
    dst = edge_index[1].astype(jnp.int32)
    h_src = h[src]
    h_dst = h[dst]
    m2 = (jnp.max(h_src) - jnp.min(h_src)) * 1.0
    attention = jnp.cos(jnp.pi / (2.0 * m2 + eps) * (h_src - h_dst))
    row_sum = jax.ops.segment_sum(attention, src, num_segments=N)
    norm_att = attention / row_sum[src] + eps
    messages = norm_att[:, None] * V2[dst]
    conv_out = jax.ops.segment_sum(messages, src, num_segments=N)
    return jax.nn.relu(former_out + conv_out)


# speedup vs baseline: 16.9264x; 14.1759x over previous
import functools

import jax
import jax.numpy as jnp
from jax import lax
from jax.experimental import pallas as pl
from jax.experimental.pallas import tpu as pltpu
from jax.experimental.pallas import tpu_sc as plsc

N = 10000
E = 320000
D = 128

NP = 10240
NW = 32
CH = E // NW
EB = 128
NBLK = E // EB
SL = NP // 16


NB = 10
BR = N // NB


HI = jax.lax.Precision.HIGHEST


def _mm(a, b):
    return jnp.dot(a, b, precision=HI)


def _ln(q):
    mu = jnp.mean(q, axis=1, keepdims=True)
    var = jnp.mean((q - mu) ** 2, axis=1, keepdims=True)
    return (q - mu) / jnp.sqrt(var + 1e-5)


def _dense_a_body(x_ref, wq_ref, bq_ref, wv_ref, bv_ref, wl_ref, bl_ref,
                  qn_ref, v_ref, v2_ref, hmax_ref, hmin_ref):
    i = pl.program_id(0)
    x = x_ref[...]
    Q = jnp.maximum(_mm(x, wq_ref[...]) + bq_ref[...], 0.0)
    v_ref[...] = _mm(x, wv_ref[...]) + bv_ref[...]
    Qn = jnp.maximum(_ln(Q), 0.0)
    qn_ref[...] = Qn
    H = jnp.sum(Qn, axis=1, keepdims=True) ** 2
    hmx = jnp.max(H).reshape(1, 1)
    hmn = jnp.min(H).reshape(1, 1)

    @pl.when(i == 0)
    def _init():
        hmax_ref[...] = hmx
        hmin_ref[...] = hmn

    @pl.when(i > 0)
    def _acc():
        hmax_ref[...] = jnp.maximum(hmax_ref[...], hmx)
        hmin_ref[...] = jnp.minimum(hmin_ref[...], hmn)
    v2_ref[...] = _mm(x, wl_ref[...]) + bl_ref[...]


def _dense_b_body(qn_ref, v_ref, hmax_ref, hmin_ref, cos_ref, sin_ref,
                  ctv_ref, stv_ref, csum_ref, ssum_ref):
    i = pl.program_id(0)
    m = (jnp.max(hmax_ref[...]) - jnp.min(hmin_ref[...])) * 2.0
    Qn = qn_ref[...]
    H = jnp.sum(Qn, axis=1, keepdims=True) ** 2
    H2 = (jnp.pi / (2.0 * m)) * H
    cosQ = Qn * jnp.cos(H2)
    sinQ = Qn * jnp.sin(H2)
    cos_ref[...] = cosQ
    sin_ref[...] = sinQ
    V = v_ref[...]
    dimn = (((0,), (0,)), ((), ()))
    ctv = jax.lax.dot_general(cosQ, V, dimn, precision=HI)
    stv = jax.lax.dot_general(sinQ, V, dimn, precision=HI)
    csum = jnp.sum(cosQ, axis=0, keepdims=True)
    ssum = jnp.sum(sinQ, axis=0, keepdims=True)

    @pl.when(i == 0)
    def _init():
        ctv_ref[...] = ctv
        stv_ref[...] = stv
        csum_ref[...] = csum
        ssum_ref[...] = ssum

    @pl.when(i > 0)
    def _acc():
        ctv_ref[...] += ctv
        stv_ref[...] += stv
        csum_ref[...] += csum
        ssum_ref[...] += ssum


def _dense_c_body(cos_ref, sin_ref, ctv_ref, stv_ref, csum_ref, ssum_ref,
                  fo_ref):
    cosQ = cos_ref[...]
    sinQ = sin_ref[...]
    out_f = _mm(cosQ, ctv_ref[...]) + _mm(sinQ, stv_ref[...])
    dimv = (((1,), (1,)), ((), ()))
    norm = (jax.lax.dot_general(cosQ, csum_ref[...], dimv, precision=HI)
            + jax.lax.dot_general(sinQ, ssum_ref[...], dimv, precision=HI) + 1e-10)
    fo_ref[...] = out_f / norm


def _row_spec(w=D):
    return pl.BlockSpec((BR, w), lambda i: (i, 0))


def _full_spec(shape):
    return pl.BlockSpec(shape, lambda i: tuple(0 for _ in shape))


@jax.jit
def _dense(x, Wq1, bq1, Wv1, bv1, Wl, bl, Wl2, bl2):
    wspec = _full_spec((D, D))
    bspec = _full_spec((1, D))
    Qn, V, V2, hmax, hmin = pl.pallas_call(
        _dense_a_body,
        grid=(NB,),
        in_specs=[_row_spec()] + [wspec, bspec] * 3,
        out_specs=(_row_spec(), _row_spec(), _row_spec(),
                   _full_spec((1, 1)), _full_spec((1, 1))),
        out_shape=(
            jax.ShapeDtypeStruct((N, D), jnp.float32),
            jax.ShapeDtypeStruct((N, D), jnp.float32),
            jax.ShapeDtypeStruct((N, D), jnp.float32),
            jax.ShapeDtypeStruct((1, 1), jnp.float32),
            jax.ShapeDtypeStruct((1, 1), jnp.float32),
        ),
    )(x, Wq1, bq1.reshape(1, D), Wv1, bv1.reshape(1, D), Wl, bl.reshape(1, D))

    cosQ, sinQ, ctv, stv, csum, ssum = pl.pallas_call(
        _dense_b_body,
        grid=(NB,),
        in_specs=[_row_spec(), _row_spec(),
                  _full_spec((1, 1)), _full_spec((1, 1))],
        out_specs=(_row_spec(), _row_spec(), _full_spec((D, D)),
                   _full_spec((D, D)), _full_spec((1, D)),
                   _full_spec((1, D))),
        out_shape=(
            jax.ShapeDtypeStruct((N, D), jnp.float32),
            jax.ShapeDtypeStruct((N, D), jnp.float32),
            jax.ShapeDtypeStruct((D, D), jnp.float32),
            jax.ShapeDtypeStruct((D, D), jnp.float32),
            jax.ShapeDtypeStruct((1, D), jnp.float32),
            jax.ShapeDtypeStruct((1, D), jnp.float32),
        ),
    )(Qn, V, hmax, hmin)

    former_out = pl.pallas_call(
        _dense_c_body,
        grid=(NB,),
        in_specs=[_row_spec(), _row_spec(), _full_spec((D, D)),
                  _full_spec((D, D)), _full_spec((1, D)),
                  _full_spec((1, D))],
        out_specs=_row_spec(),
        out_shape=jax.ShapeDtypeStruct((N, D), jnp.float32),
    )(cosQ, sinQ, ctv, stv, csum, ssum)
    return former_out, V2


def kernel(x, edge_index, Wq1, bq1, Wv1, bv1, Wl, bl, Wl2, bl2):
    eps = 1e-10
    former_out, V2 = _dense(x, Wq1, bq1, Wv1, bv1, Wl, bl, Wl2, bl2)
    Q2 = jax.nn.relu(x @ Wl2 + bl2)
    mu2 = jnp.mean(Q2, axis=-1, keepdims=True)
    var2 = jnp.var(Q2, axis=-1, keepdims=True)
    h2n = (Q2 - mu2) / jnp.sqrt(var2 + 1e-5)
    h = jnp.sum(h2n, axis=1, keepdims=True) ** 2
    h = h[:, 0]
    src = edge_index[0].astype(jnp.int32)
    dst = edge_index[1].astype(jnp.int32)
    hs, hd = _sc_gather_h(h, src, dst)
    att = _att_kernel(hs.reshape(NBLK, EB), hd.reshape(NBLK, EB)).reshape(E)
    row_sum = _sc_rowsum(src, att)
    conv = _sc_messages(src, dst, att, row_sum, V2)
    return _final_kernel(former_out, conv[0, :N], conv[1, :N])


_sc_mesh = plsc.VectorSubcoreMesh(core_axis_name="c", subcore_axis_name="s")


def _wid():
    return lax.axis_index("s") * 2 + lax.axis_index("c")


_sc_params = pltpu.CompilerParams(needs_layout_passes=False)


@functools.partial(
    pl.kernel, mesh=_sc_mesh, compiler_params=_sc_params,
    out_type=(jax.ShapeDtypeStruct((E,), jnp.float32),
              jax.ShapeDtypeStruct((E,), jnp.float32)),
    scratch_types=[pltpu.VMEM((N,), jnp.float32),
                   pltpu.VMEM((CH,), jnp.int32),
                   pltpu.VMEM((CH,), jnp.int32),
                   pltpu.VMEM((CH,), jnp.float32),
                   pltpu.VMEM((CH,), jnp.float32)],
)
def _sc_gather_h(h_hbm, src_hbm, dst_hbm, hs_hbm, hd_hbm,
                 h_v, src_v, dst_v, hs_v, hd_v):
    base = _wid() * CH
    pltpu.sync_copy(h_hbm, h_v)
    pltpu.sync_copy(src_hbm.at[pl.ds(base, CH)], src_v)
    pltpu.sync_copy(dst_hbm.at[pl.ds(base, CH)], dst_v)

    def body(i, carry):
        o = i * 16
        hs_v[pl.ds(o, 16)] = plsc.load_gather(h_v, [src_v[pl.ds(o, 16)]])
        hd_v[pl.ds(o, 16)] = plsc.load_gather(h_v, [dst_v[pl.ds(o, 16)]])
        return carry

    lax.fori_loop(0, CH // 16, body, 0)
    pltpu.sync_copy(hs_v, hs_hbm.at[pl.ds(base, CH)])
    pltpu.sync_copy(hd_v, hd_hbm.at[pl.ds(base, CH)])


def _att_body(hs_ref, hd_ref, att_ref):
    hs = hs_ref[...]
    m2 = (jnp.max(hs) - jnp.min(hs)) * 1.0
    att_ref[...] = jnp.cos(jnp.pi / (2.0 * m2 + 1e-10) * (hs - hd_ref[...]))


@jax.jit
def _att_kernel(hs, hd):
    return pl.pallas_call(
        _att_body,
        out_shape=jax.ShapeDtypeStruct((NBLK, EB), jnp.float32),
    )(hs, hd)


@functools.partial(
    pl.kernel, mesh=_sc_mesh, compiler_params=_sc_params,
    out_type=jax.ShapeDtypeStruct((2, NP), jnp.float32),
    scratch_types=[pltpu.VMEM((CH,), jnp.int32),
                   pltpu.VMEM((CH,), jnp.float32),
                   pltpu.VMEM((NP,), jnp.float32),
                   pltpu.VMEM((16, SL), jnp.float32),
                   pltpu.VMEM((SL,), jnp.float32),
                   pltpu.VMEM_SHARED((16, NP), jnp.float32)],
)
def _sc_rowsum(src_hbm, att_hbm, out_hbm,
               src_v, att_v, acc_v, red_v, res_v, shared):
    cid = lax.axis_index("c")
    sid = lax.axis_index("s")
    base = (sid * 2 + cid) * CH
    pltpu.sync_copy(src_hbm.at[pl.ds(base, CH)], src_v)
    pltpu.sync_copy(att_hbm.at[pl.ds(base, CH)], att_v)
    zero16 = jnp.zeros((16,), jnp.float32)

    def zbody(i, carry):
        acc_v[pl.ds(i * 16, 16)] = zero16
        return carry

    lax.fori_loop(0, NP // 16, zbody, 0)

    def body(i, carry):
        o = i * 16
        plsc.addupdate_scatter(acc_v, [src_v[pl.ds(o, 16)]],
                               att_v[pl.ds(o, 16)])
        return carry

    lax.fori_loop(0, CH // 16, body, 0)
    pltpu.sync_copy(acc_v, shared.at[sid])
    plsc.subcore_barrier()
    for t in range(16):
        pltpu.sync_copy(shared.at[t, pl.ds(sid * SL, SL)], red_v.at[t])

    def rbody(j, carry):
        o = j * 16
        acc = red_v[0, pl.ds(o, 16)]
        for t in range(1, 16):
            acc = acc + red_v[t, pl.ds(o, 16)]
        res_v[pl.ds(o, 16)] = acc
        return carry

    lax.fori_loop(0, SL // 16, rbody, 0)
    pltpu.sync_copy(res_v, out_hbm.at[cid, pl.ds(sid * SL, SL)])


@functools.partial(
    pl.kernel, mesh=_sc_mesh, compiler_params=_sc_params,
    out_type=jax.ShapeDtypeStruct((2, NP, D), jnp.float32),
    scratch_types=[pltpu.VMEM((2, NP), jnp.float32),
                   pltpu.VMEM((NP,), jnp.float32),
                   pltpu.VMEM((EB,), jnp.int32),
                   pltpu.VMEM((EB,), jnp.int32),
                   pltpu.VMEM((EB,), jnp.float32),
                   pltpu.VMEM((EB,), jnp.float32),
                   pltpu.VMEM((EB, D), jnp.float32),
                   pltpu.VMEM_SHARED((NP, D), jnp.float32),
                   pltpu.SemaphoreType.DMA],
)
def _sc_messages(src_hbm, dst_hbm, att_hbm, rowsum_hbm, v2_hbm, out_hbm,
                 rs2_v, rs_v, src_v, dst_v, att_v, sc_v, rows_v, acc_s, sem):
    cid = lax.axis_index("c")
    sid = lax.axis_index("s")
    w = sid * 2 + cid
    pltpu.sync_copy(rowsum_hbm, rs2_v)
    zero16 = jnp.zeros((16,), jnp.float32)

    def addb(j, carry):
        o = j * 16
        rs_v[pl.ds(o, 16)] = rs2_v[0, pl.ds(o, 16)] + rs2_v[1, pl.ds(o, 16)]
        return carry

    lax.fori_loop(0, NP // 16, addb, 0)

    def zrow(r, carry):
        for j in range(8):
            rows_v[r, pl.ds(j * 16, 16)] = zero16
        return carry

    lax.fori_loop(0, EB, zrow, 0)
    for jj in range(SL // EB):
        pltpu.sync_copy(rows_v, acc_s.at[pl.ds(sid * SL + jj * EB, EB)])
    plsc.subcore_barrier()

    nblocks = 78 + jnp.where(w < NBLK - 78 * NW, 1, 0)

    def blk_body(i, carry):
        base = (w + i * NW) * EB
        pltpu.sync_copy(src_hbm.at[pl.ds(base, EB)], src_v)
        pltpu.sync_copy(dst_hbm.at[pl.ds(base, EB)], dst_v)
        pltpu.sync_copy(att_hbm.at[pl.ds(base, EB)], att_v)
        pltpu.async_copy(v2_hbm.at[dst_v], rows_v, sem).wait()
        for j in range(8):
            o = j * 16
            rs16 = plsc.load_gather(rs_v, [src_v[pl.ds(o, 16)]])
            sc_v[pl.ds(o, 16)] = att_v[pl.ds(o, 16)] / rs16 + 1e-10

        def ebody(e, ecarry):
            idx = jnp.zeros((16,), jnp.int32) + e
            s16 = plsc.load_gather(sc_v, [idx])
            for j in range(8):
                o = j * 16
                rows_v[e, pl.ds(o, 16)] = rows_v[e, pl.ds(o, 16)] * s16
            return ecarry

        lax.fori_loop(0, EB, ebody, 0)
        pltpu.sync_copy(rows_v, acc_s.at[src_v], add=True)
        return carry

    lax.fori_loop(0, nblocks, blk_body, 0)
    plsc.subcore_barrier()
    pltpu.sync_copy(acc_s.at[pl.ds(sid * SL, SL)],
                    out_hbm.at[cid, pl.ds(sid * SL, SL)])


def _final_body(fo_ref, c0_ref, c1_ref, out_ref):
    out_ref[...] = jnp.maximum(fo_ref[...] + c0_ref[...] + c1_ref[...], 0.0)


@jax.jit
def _final_kernel(fo, c0, c1):
    return pl.pallas_call(
        _final_body,
        grid=(NB,),
        in_specs=[_row_spec(), _row_spec(), _row_spec()],
        out_specs=_row_spec(),
        out_shape=jax.ShapeDtypeStruct((N, D), jnp.float32),
    )(fo, c0, c1)
